# baseline (speedup 1.0000x reference)
import functools

import jax
import jax.numpy as jnp
from jax import lax
from jax.experimental import pallas as pl
from jax.experimental.pallas import tpu as pltpu
from jax.experimental.pallas import tpu_sc as plsc


def _sc_segment_sum(n_pad, d, nw, n_rows, r8, cap, k):
    mesh = plsc.VectorSubcoreMesh(core_axis_name="c", subcore_axis_name="s")
    rows_per_sub = n_pad // 16

    @functools.partial(
        pl.kernel,
        out_type=[
            jax.ShapeDtypeStruct((n_pad, d), jnp.float32),
            jax.ShapeDtypeStruct((n_pad, d), jnp.float32),
            jax.ShapeDtypeStruct((n_pad,), jnp.float32),
            jax.ShapeDtypeStruct((n_pad,), jnp.float32),
        ],
        mesh=mesh,
        scratch_types=[
            pltpu.VMEM((cap, k), jnp.int32),
            pltpu.VMEM((cap, k), jnp.int32),
            pltpu.VMEM((3, k, d), jnp.float32),
            pltpu.VMEM((128,), jnp.float32),
            pltpu.VMEM_SHARED((n_pad, d), jnp.float32),
            pltpu.VMEM_SHARED((n_pad,), jnp.float32),
            pltpu.SemaphoreType.DMA((3,)),
            pltpu.SemaphoreType.DMA((3,)),
            pltpu.SemaphoreType.DMA,
        ],
    )
    def sc_kernel(ei_hbm, table_hbm,
                  psum0, psum1, pcnt0, pcnt1,
                  src_v, dst_v, rows_v, ones_v, acc_s, cnt_s,
                  semg, sems, semc):
        cid = lax.axis_index("c")
        sid = lax.axis_index("s")
        wid = sid * 2 + cid

        base = ((wid * n_rows) // (8 * nw)) * 8
        nxt = jnp.where(wid == nw - 1, n_rows,
                        (((wid + 1) * n_rows) // (8 * nw)) * 8)
        n_w = nxt - base
        sb = jnp.minimum(base, r8 - cap)
        sb = pl.multiple_of(sb, 8)
        off = base - sb

        zeros16 = jnp.zeros((16,), jnp.float32)
        ones16 = jnp.ones((16,), jnp.float32)

        def zero_row(i, _):
            for j in range(d // 16):
                rows_v[0, i, pl.ds(j * 16, 16)] = zeros16
            return _

        zc = min(64, k)
        lax.fori_loop(0, zc, zero_row, None)
        for j in range(128 // 16):
            ones_v[pl.ds(j * 16, 16)] = ones16

        zbase = sid * rows_per_sub
        for j in range(rows_per_sub // zc):
            pltpu.sync_copy(rows_v.at[0, pl.ds(0, zc)],
                            acc_s.at[pl.ds(zbase + j * zc, zc)])
        for j in range(rows_per_sub // d):
            pltpu.sync_copy(rows_v.at[0, 0],
                            cnt_s.at[pl.ds(zbase + j * d, d)])

        pltpu.sync_copy(ei_hbm.at[0, 0, pl.ds(sb, cap)], src_v)
        pltpu.sync_copy(ei_hbm.at[0, 1, pl.ds(sb, cap)], dst_v)

        plsc.subcore_barrier()

        pltpu.async_copy(table_hbm.at[src_v.at[off]], rows_v.at[0],
                         semg.at[0])

        def chunk(j, _):
            b = lax.rem(j, 3)
            pltpu.make_async_copy(
                table_hbm.at[src_v.at[off + j]], rows_v.at[b],
                semg.at[b]).wait()

            @pl.when(j + 1 < n_w)
            def _():
                bn = lax.rem(j + 1, 3)
                jn = jnp.minimum(off + j + 1, cap - 1)

                @pl.when(j >= 2)
                def _():
                    pltpu.make_async_copy(
                        rows_v.at[bn], acc_s.at[dst_v.at[off]],
                        sems.at[bn]).wait()

                pltpu.async_copy(table_hbm.at[src_v.at[jn]],
                                 rows_v.at[bn], semg.at[bn])

            pltpu.async_copy(ones_v.at[pl.ds(0, k)],
                             cnt_s.at[dst_v.at[off + j]], semc, add=True)
            pltpu.async_copy(rows_v.at[b], acc_s.at[dst_v.at[off + j]],
                             sems.at[b], add=True)
            return _

        lax.fori_loop(0, n_w, chunk, None)

        for t in range(1, 4):
            @pl.when(n_w >= t)
            def _():
                bl = lax.rem(n_w - t, 3)
                pltpu.make_async_copy(rows_v.at[bl],
                                      acc_s.at[dst_v.at[off]],
                                      sems.at[bl]).wait()

        def drain(i, _):
            pltpu.make_async_copy(ones_v.at[pl.ds(0, k)],
                                  cnt_s.at[dst_v.at[off]], semc).wait()
            return _

        lax.fori_loop(0, n_w, drain, None)

        plsc.subcore_barrier()

        sl = pl.ds(zbase, rows_per_sub)

        @pl.when(cid == 0)
        def _():
            pltpu.sync_copy(acc_s.at[sl], psum0.at[sl])
            pltpu.sync_copy(cnt_s.at[sl], pcnt0.at[sl])

        @pl.when(cid == 1)
        def _():
            pltpu.sync_copy(acc_s.at[sl], psum1.at[sl])
            pltpu.sync_copy(cnt_s.at[sl], pcnt1.at[sl])

    return sc_kernel


def _tc_combine(n, d_in, d_out, rb):
    def body(x_ref, p0_ref, p1_ref, cb_ref, w1_ref, w2_ref, b_ref, o_ref):
        agg = (p0_ref[...] + p1_ref[...]) / jnp.maximum(cb_ref[...], 1.0)
        o_ref[...] = (
            jnp.dot(x_ref[...], w1_ref[...], preferred_element_type=jnp.float32)
            + jnp.dot(agg, w2_ref[...], preferred_element_type=jnp.float32)
            + b_ref[...]
        )

    grid = (n // rb,)
    return pl.pallas_call(
        body,
        grid=grid,
        in_specs=[
            pl.BlockSpec((rb, d_in), lambda i: (i, 0)),
            pl.BlockSpec((rb, d_in), lambda i: (i, 0)),
            pl.BlockSpec((rb, d_in), lambda i: (i, 0)),
            pl.BlockSpec((rb, d_in), lambda i: (i, 0)),
            pl.BlockSpec((d_in, d_out), lambda i: (0, 0)),
            pl.BlockSpec((d_in, d_out), lambda i: (0, 0)),
            pl.BlockSpec((1, d_out), lambda i: (0, 0)),
        ],
        out_specs=pl.BlockSpec((rb, d_out), lambda i: (i, 0)),
        out_shape=jax.ShapeDtypeStruct((n, d_out), jnp.float32),
    )


@jax.jit
def kernel(node_features, edge_indices, W, b):
    bsz, n, d = node_features.shape
    e = edge_indices.shape[-1]
    d_out = W.shape[0]

    nw = 32
    k = 128
    for cand in range(80, 63, -1):
        if e % cand == 0 and (e // cand) % 8 == 0 and e // cand >= nw:
            k = cand
            break
    n_pad = -(-(n + 64) // 2048) * 2048

    rb = 1
    for cand in range(8, min(n, 1024) + 1, 8):
        if n % cand == 0:
            rb = cand
    if rb == 1:
        rb = n

    n_rows = -(-e // k)
    r8 = -(-n_rows // 8) * 8
    bases = [((w * n_rows) // (8 * nw)) * 8 for w in range(nw)] + [n_rows]
    cap = -(-max(b1 - b0 for b0, b1 in zip(bases, bases[1:])) // 8) * 8
    cap = min(max(cap, 8), r8)
    sc_fn = _sc_segment_sum(n_pad, d, nw, n_rows, r8, cap, k)
    tc_fn = _tc_combine(n, d, d_out, rb)

    w1t = jnp.transpose(W[:, :d])
    w2t = jnp.transpose(W[:, d:])
    b2 = b.reshape(1, d_out)

    outs = []
    for bi in range(bsz):
        ei = edge_indices[bi]
        pad = r8 * k - e
        if pad:
            pad_dst = n + jnp.arange(pad, dtype=jnp.int32) % (n_pad - n)
            ei = jnp.concatenate(
                [ei, jnp.stack([jnp.zeros((pad,), jnp.int32), pad_dst])],
                axis=1)
        ei4 = ei.reshape(1, 2, r8, k)

        x = node_features[bi]
        psum0, psum1, pcnt0, pcnt1 = sc_fn(ei4, x)
        cnt_b = jnp.broadcast_to((pcnt0 + pcnt1)[:, None], (n_pad, d))
        out = tc_fn(x, psum0, psum1, cnt_b, w1t, w2t, b2)
        outs.append(out[None])
    return jnp.concatenate(outs, axis=0) if bsz > 1 else outs[0]


# speedup vs baseline: 1.1183x; 1.1183x over previous
import functools

import jax
import jax.numpy as jnp
from jax import lax
from jax.experimental import pallas as pl
from jax.experimental.pallas import tpu as pltpu
from jax.experimental.pallas import tpu_sc as plsc


def _sc_segment_sum(n_pad, d, nw, n_rows, r8, cap, k):
    mesh = plsc.VectorSubcoreMesh(core_axis_name="c", subcore_axis_name="s")
    rows_per_sub = n_pad // 16

    @functools.partial(
        pl.kernel,
        out_type=[
            jax.ShapeDtypeStruct((n_pad, d), jnp.float32),
            jax.ShapeDtypeStruct((n_pad, d), jnp.float32),
            jax.ShapeDtypeStruct((n_pad,), jnp.float32),
            jax.ShapeDtypeStruct((n_pad,), jnp.float32),
        ],
        mesh=mesh,
        scratch_types=[
            pltpu.VMEM((cap, k), jnp.int32),
            pltpu.VMEM((cap, k), jnp.int32),
            pltpu.VMEM((2, k, d), jnp.float32),
            pltpu.VMEM((128,), jnp.float32),
            pltpu.VMEM_SHARED((n_pad, d), jnp.float32),
            pltpu.VMEM_SHARED((n_pad,), jnp.float32),
            pltpu.SemaphoreType.DMA((2,)),
            pltpu.SemaphoreType.DMA,
        ],
    )
    def sc_kernel(ei_hbm, table_hbm,
                  psum0, psum1, pcnt0, pcnt1,
                  src_v, dst_v, rows_v, ones_v, acc_s, cnt_s, semg, semc):
        cid = lax.axis_index("c")
        sid = lax.axis_index("s")
        wid = sid * 2 + cid

        base = (wid * n_rows) // nw
        nxt = ((wid + 1) * n_rows) // nw
        n_w = nxt - base
        sb = jnp.minimum((base // 8) * 8, r8 - cap)
        sb = pl.multiple_of(sb, 8)
        off = base - sb

        zeros16 = jnp.zeros((16,), jnp.float32)
        ones16 = jnp.ones((16,), jnp.float32)

        def zero_row(i, _):
            for j in range(d // 16):
                rows_v[0, i, pl.ds(j * 16, 16)] = zeros16
            return _

        zc = min(64, k)
        lax.fori_loop(0, zc, zero_row, None)
        for j in range(128 // 16):
            ones_v[pl.ds(j * 16, 16)] = ones16

        zbase = sid * rows_per_sub
        for j in range(rows_per_sub // zc):
            pltpu.sync_copy(rows_v.at[0, pl.ds(0, zc)],
                            acc_s.at[pl.ds(zbase + j * zc, zc)])
        for j in range(rows_per_sub // d):
            pltpu.sync_copy(rows_v.at[0, 0],
                            cnt_s.at[pl.ds(zbase + j * d, d)])

        pltpu.sync_copy(ei_hbm.at[0, 0, pl.ds(sb, cap)], src_v)
        pltpu.sync_copy(ei_hbm.at[0, 1, pl.ds(sb, cap)], dst_v)

        plsc.subcore_barrier()

        pltpu.async_copy(table_hbm.at[src_v.at[off]], rows_v.at[0],
                         semg.at[0])

        def chunk(j, _):
            b = lax.rem(j, 2)
            pltpu.make_async_copy(
                table_hbm.at[src_v.at[off + j]], rows_v.at[b],
                semg.at[b]).wait()

            @pl.when(j + 1 < n_w)
            def _():
                jn = jnp.minimum(off + j + 1, cap - 1)
                pltpu.async_copy(table_hbm.at[src_v.at[jn]],
                                 rows_v.at[1 - b], semg.at[1 - b])

            pltpu.async_copy(ones_v.at[pl.ds(0, k)],
                             cnt_s.at[dst_v.at[off + j]], semc, add=True)
            pltpu.sync_copy(rows_v.at[b], acc_s.at[dst_v.at[off + j]],
                            add=True)
            return _

        lax.fori_loop(0, n_w, chunk, None)

        def drain(i, _):
            pltpu.make_async_copy(ones_v.at[pl.ds(0, k)],
                                  cnt_s.at[dst_v.at[off]], semc).wait()
            return _

        lax.fori_loop(0, n_w, drain, None)

        plsc.subcore_barrier()

        sl = pl.ds(zbase, rows_per_sub)

        @pl.when(cid == 0)
        def _():
            pltpu.sync_copy(acc_s.at[sl], psum0.at[sl])
            pltpu.sync_copy(cnt_s.at[sl], pcnt0.at[sl])

        @pl.when(cid == 1)
        def _():
            pltpu.sync_copy(acc_s.at[sl], psum1.at[sl])
            pltpu.sync_copy(cnt_s.at[sl], pcnt1.at[sl])

    return sc_kernel


def _tc_dense(n, d_in, d_out, rb):
    def body(x_ref, w1_ref, b_ref, o_ref):
        o_ref[...] = jnp.dot(
            x_ref[...], w1_ref[...],
            preferred_element_type=jnp.float32) + b_ref[...]

    return pl.pallas_call(
        body,
        grid=(n // rb,),
        in_specs=[
            pl.BlockSpec((rb, d_in), lambda i: (i, 0)),
            pl.BlockSpec((d_in, d_out), lambda i: (0, 0)),
            pl.BlockSpec((1, d_out), lambda i: (0, 0)),
        ],
        out_specs=pl.BlockSpec((rb, d_out), lambda i: (i, 0)),
        out_shape=jax.ShapeDtypeStruct((n, d_out), jnp.float32),
    )


def _tc_combine(n, d_in, d_out, rb):
    def body(y1_ref, p0_ref, p1_ref, cb_ref, w2_ref, o_ref):
        cnt = jnp.maximum(cb_ref[...].astype(jnp.float32), 1.0)
        agg = (p0_ref[...] + p1_ref[...]) / cnt
        o_ref[...] = y1_ref[...] + jnp.dot(
            agg, w2_ref[...], preferred_element_type=jnp.float32)

    grid = (n // rb,)
    return pl.pallas_call(
        body,
        grid=grid,
        in_specs=[
            pl.BlockSpec((rb, d_out), lambda i: (i, 0)),
            pl.BlockSpec((rb, d_in), lambda i: (i, 0)),
            pl.BlockSpec((rb, d_in), lambda i: (i, 0)),
            pl.BlockSpec((rb, d_in), lambda i: (i, 0)),
            pl.BlockSpec((d_in, d_out), lambda i: (0, 0)),
        ],
        out_specs=pl.BlockSpec((rb, d_out), lambda i: (i, 0)),
        out_shape=jax.ShapeDtypeStruct((n, d_out), jnp.float32),
    )


@jax.jit
def kernel(node_features, edge_indices, W, b):
    bsz, n, d = node_features.shape
    e = edge_indices.shape[-1]
    d_out = W.shape[0]

    nw = 32
    k = 128
    for cand in range(128, 63, -1):
        if e % cand == 0 and (e // cand) % 8 == 0 and e // cand >= nw:
            k = cand
            break
    n_pad = -(-(n + 64) // 2048) * 2048

    rb = 1
    for cand in range(8, min(n, 1024) + 1, 8):
        if n % cand == 0:
            rb = cand
    if rb == 1:
        rb = n

    n_rows = -(-e // k)
    r8 = -(-n_rows // 8) * 8
    cap = -(-((-(-n_rows // nw) + 1) + 8) // 8) * 8
    cap = min(cap, r8)
    sc_fn = _sc_segment_sum(n_pad, d, nw, n_rows, r8, cap, k)
    tc1_fn = _tc_dense(n, d, d_out, rb)
    tc2_fn = _tc_combine(n, d, d_out, rb)

    w1t = jnp.transpose(W[:, :d])
    w2t = jnp.transpose(W[:, d:])
    b2 = b.reshape(1, d_out)

    outs = []
    for bi in range(bsz):
        ei = edge_indices[bi]
        pad = r8 * k - e
        if pad:
            pad_dst = n + jnp.arange(pad, dtype=jnp.int32) % (n_pad - n)
            ei = jnp.concatenate(
                [ei, jnp.stack([jnp.zeros((pad,), jnp.int32), pad_dst])],
                axis=1)
        ei4 = ei.reshape(1, 2, r8, k)

        x = node_features[bi]
        psum0, psum1, pcnt0, pcnt1 = sc_fn(ei4, x)
        y1 = tc1_fn(x, w1t, b2)
        cnt_b = jnp.broadcast_to(
            (pcnt0 + pcnt1).astype(jnp.bfloat16)[:, None], (n_pad, d))
        out = tc2_fn(y1, psum0, psum1, cnt_b, w2t)
        outs.append(out[None])
    return jnp.concatenate(outs, axis=0) if bsz > 1 else outs[0]


# speedup vs baseline: 1.1917x; 1.0657x over previous
import functools

import jax
import jax.numpy as jnp
from jax import lax
from jax.experimental import pallas as pl
from jax.experimental.pallas import tpu as pltpu
from jax.experimental.pallas import tpu_sc as plsc


def _sc_segment_sum(n_pad, d, nw, n_rows, r8, cap, k):
    mesh = plsc.VectorSubcoreMesh(core_axis_name="c", subcore_axis_name="s")
    rows_per_sub = n_pad // 16

    @functools.partial(
        pl.kernel,
        out_type=[
            jax.ShapeDtypeStruct((n_pad, d), jnp.float32),
            jax.ShapeDtypeStruct((n_pad, d), jnp.float32),
            jax.ShapeDtypeStruct((n_pad,), jnp.float32),
            jax.ShapeDtypeStruct((n_pad,), jnp.float32),
        ],
        mesh=mesh,
        scratch_types=[
            pltpu.VMEM((cap, k), jnp.int32),
            pltpu.VMEM((cap, k), jnp.int32),
            pltpu.VMEM((2, k, d), jnp.float32),
            pltpu.VMEM((128,), jnp.float32),
            pltpu.VMEM_SHARED((n_pad, d), jnp.float32),
            pltpu.VMEM_SHARED((n_pad,), jnp.float32),
            pltpu.SemaphoreType.DMA((2,)),
            pltpu.SemaphoreType.DMA,
        ],
    )
    def sc_kernel(ei_hbm, table_hbm,
                  psum0, psum1, pcnt0, pcnt1,
                  src_v, dst_v, rows_v, ones_v, acc_s, cnt_s, semg, semc):
        cid = lax.axis_index("c")
        sid = lax.axis_index("s")
        wid = sid * 2 + cid

        base = (wid * n_rows) // nw
        nxt = ((wid + 1) * n_rows) // nw
        n_w = nxt - base
        sb = jnp.minimum((base // 8) * 8, r8 - cap)
        sb = pl.multiple_of(sb, 8)
        off = base - sb

        zeros16 = jnp.zeros((16,), jnp.float32)
        ones16 = jnp.ones((16,), jnp.float32)

        def zero_row(i, _):
            for j in range(d // 16):
                rows_v[0, i, pl.ds(j * 16, 16)] = zeros16
            return _

        zc = min(64, k)
        lax.fori_loop(0, zc, zero_row, None)
        for j in range(128 // 16):
            ones_v[pl.ds(j * 16, 16)] = ones16

        zbase = sid * rows_per_sub
        for j in range(rows_per_sub // zc):
            pltpu.sync_copy(rows_v.at[0, pl.ds(0, zc)],
                            acc_s.at[pl.ds(zbase + j * zc, zc)])
        for j in range(rows_per_sub // d):
            pltpu.sync_copy(rows_v.at[0, 0],
                            cnt_s.at[pl.ds(zbase + j * d, d)])

        pltpu.sync_copy(ei_hbm.at[0, 0, pl.ds(sb, cap)], src_v)
        pltpu.sync_copy(ei_hbm.at[0, 1, pl.ds(sb, cap)], dst_v)

        plsc.subcore_barrier()

        pltpu.async_copy(table_hbm.at[src_v.at[off]], rows_v.at[0],
                         semg.at[0])

        def chunk(j, _):
            b = lax.rem(j, 2)
            pltpu.make_async_copy(
                table_hbm.at[src_v.at[off + j]], rows_v.at[b],
                semg.at[b]).wait()

            @pl.when(j + 1 < n_w)
            def _():
                jn = jnp.minimum(off + j + 1, cap - 1)
                pltpu.async_copy(table_hbm.at[src_v.at[jn]],
                                 rows_v.at[1 - b], semg.at[1 - b])

            pltpu.async_copy(ones_v.at[pl.ds(0, k)],
                             cnt_s.at[dst_v.at[off + j]], semc, add=True)
            pltpu.sync_copy(rows_v.at[b], acc_s.at[dst_v.at[off + j]],
                            add=True)
            return _

        lax.fori_loop(0, n_w, chunk, None)

        def drain(i, _):
            pltpu.make_async_copy(ones_v.at[pl.ds(0, k)],
                                  cnt_s.at[dst_v.at[off]], semc).wait()
            return _

        lax.fori_loop(0, n_w, drain, None)

        plsc.subcore_barrier()

        sl = pl.ds(zbase, rows_per_sub)

        @pl.when(cid == 0)
        def _():
            pltpu.sync_copy(acc_s.at[sl], psum0.at[sl])
            pltpu.sync_copy(cnt_s.at[sl], pcnt0.at[sl])

        @pl.when(cid == 1)
        def _():
            pltpu.sync_copy(acc_s.at[sl], psum1.at[sl])
            pltpu.sync_copy(cnt_s.at[sl], pcnt1.at[sl])

    return sc_kernel


def _tc_dense(n, d_in, d_out, rb):
    def body(x_ref, w1_ref, b_ref, o_ref):
        o_ref[...] = jnp.dot(
            x_ref[...], w1_ref[...],
            preferred_element_type=jnp.float32) + b_ref[...]

    return pl.pallas_call(
        body,
        grid=(n // rb,),
        in_specs=[
            pl.BlockSpec((rb, d_in), lambda i: (i, 0)),
            pl.BlockSpec((d_in, d_out), lambda i: (0, 0)),
            pl.BlockSpec((1, d_out), lambda i: (0, 0)),
        ],
        out_specs=pl.BlockSpec((rb, d_out), lambda i: (i, 0)),
        out_shape=jax.ShapeDtypeStruct((n, d_out), jnp.float32),
    )


def _tc_combine(n, d_in, d_out):
    rb = 1024

    def body(y1_ref, p0_ref, p1_ref, c0_ref, c1_ref, w2_ref, o_ref):
        c8 = c0_ref[...] + c1_ref[...]
        row = lax.broadcasted_iota(jnp.int32, (128, 128), 0)
        col = lax.broadcasted_iota(jnp.int32, (128, 128), 1)
        eye = (row == col).astype(jnp.float32)
        for g in range(rb // 128):
            sl = pl.ds(g * 128, 128)
            m = jnp.broadcast_to(c8[g:g + 1, :], (128, 128))
            cnt = jnp.sum(m * eye, axis=-1, keepdims=True)
            agg = (p0_ref[sl, :] + p1_ref[sl, :]) / jnp.maximum(cnt, 1.0)
            o_ref[sl, :] = y1_ref[sl, :] + jnp.dot(
                agg, w2_ref[...], preferred_element_type=jnp.float32)

    grid = (-(-n // rb),)
    return pl.pallas_call(
        body,
        grid=grid,
        in_specs=[
            pl.BlockSpec((rb, d_out), lambda i: (i, 0)),
            pl.BlockSpec((rb, d_in), lambda i: (i, 0)),
            pl.BlockSpec((rb, d_in), lambda i: (i, 0)),
            pl.BlockSpec((rb // 128, 128), lambda i: (i, 0)),
            pl.BlockSpec((rb // 128, 128), lambda i: (i, 0)),
            pl.BlockSpec((d_in, d_out), lambda i: (0, 0)),
        ],
        out_specs=pl.BlockSpec((rb, d_out), lambda i: (i, 0)),
        out_shape=jax.ShapeDtypeStruct((n, d_out), jnp.float32),
    )


@jax.jit
def kernel(node_features, edge_indices, W, b):
    bsz, n, d = node_features.shape
    e = edge_indices.shape[-1]
    d_out = W.shape[0]

    nw = 32
    k = 128
    for cand in range(128, 63, -1):
        if e % cand == 0 and (e // cand) % 8 == 0 and e // cand >= nw:
            k = cand
            break
    n_pad = -(-(n + 64) // 2048) * 2048

    rb = 1
    for cand in range(8, min(n, 1024) + 1, 8):
        if n % cand == 0:
            rb = cand
    if rb == 1:
        rb = n

    n_rows = -(-e // k)
    r8 = -(-n_rows // 8) * 8
    cap = -(-((-(-n_rows // nw) + 1) + 8) // 8) * 8
    cap = min(cap, r8)
    sc_fn = _sc_segment_sum(n_pad, d, nw, n_rows, r8, cap, k)
    tc1_fn = _tc_dense(n, d, d_out, rb)
    tc2_fn = _tc_combine(n, d, d_out)

    w1t = jnp.transpose(W[:, :d])
    w2t = jnp.transpose(W[:, d:])
    b2 = b.reshape(1, d_out)

    outs = []
    for bi in range(bsz):
        ei = edge_indices[bi]
        pad = r8 * k - e
        if pad:
            pad_dst = n + jnp.arange(pad, dtype=jnp.int32) % (n_pad - n)
            ei = jnp.concatenate(
                [ei, jnp.stack([jnp.zeros((pad,), jnp.int32), pad_dst])],
                axis=1)
        ei4 = ei.reshape(1, 2, r8, k)

        x = node_features[bi]
        psum0, psum1, pcnt0, pcnt1 = sc_fn(ei4, x)
        y1 = tc1_fn(x, w1t, b2)
        out = tc2_fn(y1, psum0, psum1,
                     pcnt0.reshape(n_pad // 128, 128),
                     pcnt1.reshape(n_pad // 128, 128), w2t)
        outs.append(out[None])
    return jnp.concatenate(outs, axis=0) if bsz > 1 else outs[0]


# speedup vs baseline: 1.2241x; 1.0271x over previous
import functools

import jax
import jax.numpy as jnp
from jax import lax
from jax.experimental import pallas as pl
from jax.experimental.pallas import tpu as pltpu
from jax.experimental.pallas import tpu_sc as plsc


def _sc_segment_sum(n_pad, d, nw, n_rows, r8, cap, k):
    mesh = plsc.VectorSubcoreMesh(core_axis_name="c", subcore_axis_name="s")
    rows_per_sub = n_pad // 16

    @functools.partial(
        pl.kernel,
        out_type=[
            jax.ShapeDtypeStruct((n_pad, d), jnp.float32),
            jax.ShapeDtypeStruct((n_pad, d), jnp.float32),
            jax.ShapeDtypeStruct((n_pad,), jnp.float32),
            jax.ShapeDtypeStruct((n_pad,), jnp.float32),
        ],
        mesh=mesh,
        scratch_types=[
            pltpu.VMEM((cap, k), jnp.int32),
            pltpu.VMEM((cap, k), jnp.int32),
            pltpu.VMEM((2, k, d), jnp.float32),
            pltpu.VMEM((128,), jnp.float32),
            pltpu.VMEM_SHARED((n_pad, d), jnp.float32),
            pltpu.VMEM_SHARED((n_pad,), jnp.float32),
            pltpu.SemaphoreType.DMA((2,)),
            pltpu.SemaphoreType.DMA,
        ],
    )
    def sc_kernel(ei_hbm, table_hbm,
                  psum0, psum1, pcnt0, pcnt1,
                  src_v, dst_v, rows_v, ones_v, acc_s, cnt_s, semg, semc):
        cid = lax.axis_index("c")
        sid = lax.axis_index("s")
        wid = sid * 2 + cid

        base = (wid * n_rows) // nw
        nxt = ((wid + 1) * n_rows) // nw
        n_w = nxt - base
        sb = jnp.minimum((base // 8) * 8, r8 - cap)
        sb = pl.multiple_of(sb, 8)
        off = base - sb

        zeros16 = jnp.zeros((16,), jnp.float32)
        ones16 = jnp.ones((16,), jnp.float32)

        def zero_row(i, _):
            for j in range(d // 16):
                rows_v[0, i, pl.ds(j * 16, 16)] = zeros16
            return _

        zc = min(64, k)
        lax.fori_loop(0, zc, zero_row, None)
        for j in range(128 // 16):
            ones_v[pl.ds(j * 16, 16)] = ones16

        zbase = sid * rows_per_sub
        for j in range(rows_per_sub // zc):
            pltpu.async_copy(rows_v.at[0, pl.ds(0, zc)],
                             acc_s.at[pl.ds(zbase + j * zc, zc)], semc)
        for j in range(rows_per_sub // d):
            pltpu.async_copy(rows_v.at[0, 0],
                             cnt_s.at[pl.ds(zbase + j * d, d)], semc)

        pltpu.async_copy(ei_hbm.at[0, 0, pl.ds(sb, cap)], src_v, semg.at[0])
        pltpu.async_copy(ei_hbm.at[0, 1, pl.ds(sb, cap)], dst_v, semg.at[1])

        for j in range(rows_per_sub // zc):
            pltpu.make_async_copy(
                rows_v.at[0, pl.ds(0, zc)],
                acc_s.at[pl.ds(zbase + j * zc, zc)], semc).wait()
        for j in range(rows_per_sub // d):
            pltpu.make_async_copy(
                rows_v.at[0, 0],
                cnt_s.at[pl.ds(zbase + j * d, d)], semc).wait()
        pltpu.make_async_copy(
            ei_hbm.at[0, 0, pl.ds(sb, cap)], src_v, semg.at[0]).wait()
        pltpu.make_async_copy(
            ei_hbm.at[0, 1, pl.ds(sb, cap)], dst_v, semg.at[1]).wait()

        pltpu.async_copy(table_hbm.at[src_v.at[off]], rows_v.at[0],
                         semg.at[0])

        plsc.subcore_barrier()

        def chunk(j, _):
            b = lax.rem(j, 2)
            pltpu.make_async_copy(
                table_hbm.at[src_v.at[off + j]], rows_v.at[b],
                semg.at[b]).wait()

            @pl.when(j + 1 < n_w)
            def _():
                jn = jnp.minimum(off + j + 1, cap - 1)
                pltpu.async_copy(table_hbm.at[src_v.at[jn]],
                                 rows_v.at[1 - b], semg.at[1 - b])

            pltpu.async_copy(ones_v.at[pl.ds(0, k)],
                             cnt_s.at[dst_v.at[off + j]], semc, add=True)
            pltpu.sync_copy(rows_v.at[b], acc_s.at[dst_v.at[off + j]],
                            add=True)
            return _

        lax.fori_loop(0, n_w, chunk, None)

        def drain(i, _):
            pltpu.make_async_copy(ones_v.at[pl.ds(0, k)],
                                  cnt_s.at[dst_v.at[off]], semc).wait()
            return _

        lax.fori_loop(0, n_w, drain, None)

        plsc.subcore_barrier()

        sl = pl.ds(zbase, rows_per_sub)

        @pl.when(cid == 0)
        def _():
            pltpu.sync_copy(acc_s.at[sl], psum0.at[sl])
            pltpu.sync_copy(cnt_s.at[sl], pcnt0.at[sl])

        @pl.when(cid == 1)
        def _():
            pltpu.sync_copy(acc_s.at[sl], psum1.at[sl])
            pltpu.sync_copy(cnt_s.at[sl], pcnt1.at[sl])

    return sc_kernel


def _tc_dense(n, d_in, d_out, rb):
    def body(x_ref, w1_ref, b_ref, o_ref):
        o_ref[...] = jnp.dot(
            x_ref[...], w1_ref[...],
            preferred_element_type=jnp.float32) + b_ref[...]

    return pl.pallas_call(
        body,
        grid=(n // rb,),
        in_specs=[
            pl.BlockSpec((rb, d_in), lambda i: (i, 0)),
            pl.BlockSpec((d_in, d_out), lambda i: (0, 0)),
            pl.BlockSpec((1, d_out), lambda i: (0, 0)),
        ],
        out_specs=pl.BlockSpec((rb, d_out), lambda i: (i, 0)),
        out_shape=jax.ShapeDtypeStruct((n, d_out), jnp.float32),
    )


def _tc_combine(n, d_in, d_out):
    rb = 1024

    def body(y1_ref, p0_ref, p1_ref, c0_ref, c1_ref, w2_ref, o_ref):
        c8 = c0_ref[...] + c1_ref[...]
        row = lax.broadcasted_iota(jnp.int32, (128, 128), 0)
        col = lax.broadcasted_iota(jnp.int32, (128, 128), 1)
        eye = (row == col).astype(jnp.float32)
        for g in range(rb // 128):
            sl = pl.ds(g * 128, 128)
            m = jnp.broadcast_to(c8[g:g + 1, :], (128, 128))
            cnt = jnp.sum(m * eye, axis=-1, keepdims=True)
            agg = (p0_ref[sl, :] + p1_ref[sl, :]) / jnp.maximum(cnt, 1.0)
            o_ref[sl, :] = y1_ref[sl, :] + jnp.dot(
                agg, w2_ref[...], preferred_element_type=jnp.float32)

    grid = (-(-n // rb),)
    return pl.pallas_call(
        body,
        grid=grid,
        in_specs=[
            pl.BlockSpec((rb, d_out), lambda i: (i, 0)),
            pl.BlockSpec((rb, d_in), lambda i: (i, 0)),
            pl.BlockSpec((rb, d_in), lambda i: (i, 0)),
            pl.BlockSpec((rb // 128, 128), lambda i: (i, 0)),
            pl.BlockSpec((rb // 128, 128), lambda i: (i, 0)),
            pl.BlockSpec((d_in, d_out), lambda i: (0, 0)),
        ],
        out_specs=pl.BlockSpec((rb, d_out), lambda i: (i, 0)),
        out_shape=jax.ShapeDtypeStruct((n, d_out), jnp.float32),
    )


@jax.jit
def kernel(node_features, edge_indices, W, b):
    bsz, n, d = node_features.shape
    e = edge_indices.shape[-1]
    d_out = W.shape[0]

    nw = 32
    k = 128
    for cand in range(128, 63, -1):
        if e % cand == 0 and (e // cand) % 8 == 0 and e // cand >= nw:
            k = cand
            break
    n_pad = -(-(n + 64) // 2048) * 2048

    rb = 1
    for cand in range(8, min(n, 1024) + 1, 8):
        if n % cand == 0:
            rb = cand
    if rb == 1:
        rb = n

    n_rows = -(-e // k)
    r8 = -(-n_rows // 8) * 8
    cap = -(-((-(-n_rows // nw) + 1) + 8) // 8) * 8
    cap = min(cap, r8)
    sc_fn = _sc_segment_sum(n_pad, d, nw, n_rows, r8, cap, k)
    tc1_fn = _tc_dense(n, d, d_out, rb)
    tc2_fn = _tc_combine(n, d, d_out)

    w1t = jnp.transpose(W[:, :d])
    w2t = jnp.transpose(W[:, d:])
    b2 = b.reshape(1, d_out)

    outs = []
    for bi in range(bsz):
        ei = edge_indices[bi]
        pad = r8 * k - e
        if pad:
            pad_dst = n + jnp.arange(pad, dtype=jnp.int32) % (n_pad - n)
            ei = jnp.concatenate(
                [ei, jnp.stack([jnp.zeros((pad,), jnp.int32), pad_dst])],
                axis=1)
        ei4 = ei.reshape(1, 2, r8, k)

        x = node_features[bi]
        psum0, psum1, pcnt0, pcnt1 = sc_fn(ei4, x)
        y1 = tc1_fn(x, w1t, b2)
        out = tc2_fn(y1, psum0, psum1,
                     pcnt0.reshape(n_pad // 128, 128),
                     pcnt1.reshape(n_pad // 128, 128), w2t)
        outs.append(out[None])
    return jnp.concatenate(outs, axis=0) if bsz > 1 else outs[0]


# speedup vs baseline: 1.2283x; 1.0034x over previous
import functools

import jax
import jax.numpy as jnp
from jax import lax
from jax.experimental import pallas as pl
from jax.experimental.pallas import tpu as pltpu
from jax.experimental.pallas import tpu_sc as plsc


def _sc_segment_sum(n_pad, d, nw, n_rows, r8, cap, k):
    mesh = plsc.VectorSubcoreMesh(core_axis_name="c", subcore_axis_name="s")
    rows_per_sub = n_pad // 16

    @functools.partial(
        pl.kernel,
        out_type=[
            jax.ShapeDtypeStruct((n_pad, d), jnp.float32),
            jax.ShapeDtypeStruct((n_pad, d), jnp.float32),
            jax.ShapeDtypeStruct((n_pad,), jnp.float32),
            jax.ShapeDtypeStruct((n_pad,), jnp.float32),
        ],
        mesh=mesh,
        scratch_types=[
            pltpu.VMEM((cap, k), jnp.int32),
            pltpu.VMEM((cap, k), jnp.int32),
            pltpu.VMEM((2, k, d), jnp.float32),
            pltpu.VMEM((128,), jnp.float32),
            pltpu.VMEM_SHARED((n_pad, d), jnp.float32),
            pltpu.VMEM_SHARED((n_pad,), jnp.float32),
            pltpu.SemaphoreType.DMA((2,)),
            pltpu.SemaphoreType.DMA,
        ],
    )
    def sc_kernel(ei_hbm, table_hbm,
                  psum0, psum1, pcnt0, pcnt1,
                  src_v, dst_v, rows_v, ones_v, acc_s, cnt_s, semg, semc):
        cid = lax.axis_index("c")
        sid = lax.axis_index("s")
        wid = sid * 2 + cid

        base = (wid * n_rows) // nw
        nxt = ((wid + 1) * n_rows) // nw
        n_w = nxt - base
        sb = jnp.minimum((base // 8) * 8, r8 - cap)
        sb = pl.multiple_of(sb, 8)
        off = base - sb

        zeros16 = jnp.zeros((16,), jnp.float32)
        ones16 = jnp.ones((16,), jnp.float32)

        def zero_row(i, _):
            for j in range(d // 16):
                rows_v[0, i, pl.ds(j * 16, 16)] = zeros16
            return _

        zc = min(64, k)
        lax.fori_loop(0, zc, zero_row, None)
        for j in range(128 // 16):
            ones_v[pl.ds(j * 16, 16)] = ones16

        zbase = sid * rows_per_sub
        for j in range(rows_per_sub // zc):
            pltpu.async_copy(rows_v.at[0, pl.ds(0, zc)],
                             acc_s.at[pl.ds(zbase + j * zc, zc)], semc)
        for j in range(rows_per_sub // d):
            pltpu.async_copy(rows_v.at[0, 0],
                             cnt_s.at[pl.ds(zbase + j * d, d)], semc)

        pltpu.async_copy(ei_hbm.at[0, 0, pl.ds(sb, cap)], src_v, semg.at[0])
        pltpu.async_copy(ei_hbm.at[0, 1, pl.ds(sb, cap)], dst_v, semg.at[1])

        for j in range(rows_per_sub // zc):
            pltpu.make_async_copy(
                rows_v.at[0, pl.ds(0, zc)],
                acc_s.at[pl.ds(zbase + j * zc, zc)], semc).wait()
        for j in range(rows_per_sub // d):
            pltpu.make_async_copy(
                rows_v.at[0, 0],
                cnt_s.at[pl.ds(zbase + j * d, d)], semc).wait()
        pltpu.make_async_copy(
            ei_hbm.at[0, 0, pl.ds(sb, cap)], src_v, semg.at[0]).wait()
        pltpu.make_async_copy(
            ei_hbm.at[0, 1, pl.ds(sb, cap)], dst_v, semg.at[1]).wait()

        pltpu.async_copy(table_hbm.at[src_v.at[off]], rows_v.at[0],
                         semg.at[0])

        plsc.subcore_barrier()

        def chunk(j, _):
            b = lax.rem(j, 2)
            pltpu.make_async_copy(
                table_hbm.at[src_v.at[off + j]], rows_v.at[b],
                semg.at[b]).wait()

            @pl.when(j + 1 < n_w)
            def _():
                jn = jnp.minimum(off + j + 1, cap - 1)
                pltpu.async_copy(table_hbm.at[src_v.at[jn]],
                                 rows_v.at[1 - b], semg.at[1 - b])

            pltpu.async_copy(ones_v.at[pl.ds(0, k)],
                             cnt_s.at[dst_v.at[off + j]], semc, add=True)
            pltpu.sync_copy(rows_v.at[b], acc_s.at[dst_v.at[off + j]],
                            add=True)
            return _

        lax.fori_loop(0, n_w, chunk, None)

        def drain(i, _):
            pltpu.make_async_copy(ones_v.at[pl.ds(0, k)],
                                  cnt_s.at[dst_v.at[off]], semc).wait()
            return _

        lax.fori_loop(0, n_w, drain, None)

        plsc.subcore_barrier()

        sl = pl.ds(zbase, rows_per_sub)

        @pl.when(cid == 0)
        def _():
            pltpu.async_copy(acc_s.at[sl], psum0.at[sl], semg.at[0])
            pltpu.async_copy(cnt_s.at[sl], pcnt0.at[sl], semg.at[1])
            pltpu.make_async_copy(acc_s.at[sl], psum0.at[sl],
                                  semg.at[0]).wait()
            pltpu.make_async_copy(cnt_s.at[sl], pcnt0.at[sl],
                                  semg.at[1]).wait()

        @pl.when(cid == 1)
        def _():
            pltpu.async_copy(acc_s.at[sl], psum1.at[sl], semg.at[0])
            pltpu.async_copy(cnt_s.at[sl], pcnt1.at[sl], semg.at[1])
            pltpu.make_async_copy(acc_s.at[sl], psum1.at[sl],
                                  semg.at[0]).wait()
            pltpu.make_async_copy(cnt_s.at[sl], pcnt1.at[sl],
                                  semg.at[1]).wait()

    return sc_kernel


def _tc_dense(n, d_in, d_out, rb):
    def body(x_ref, w1_ref, b_ref, o_ref):
        o_ref[...] = jnp.dot(
            x_ref[...], w1_ref[...],
            preferred_element_type=jnp.float32) + b_ref[...]

    return pl.pallas_call(
        body,
        grid=(n // rb,),
        in_specs=[
            pl.BlockSpec((rb, d_in), lambda i: (i, 0)),
            pl.BlockSpec((d_in, d_out), lambda i: (0, 0)),
            pl.BlockSpec((1, d_out), lambda i: (0, 0)),
        ],
        out_specs=pl.BlockSpec((rb, d_out), lambda i: (i, 0)),
        out_shape=jax.ShapeDtypeStruct((n, d_out), jnp.float32),
    )


def _tc_combine(n, d_in, d_out):
    rb = 1024

    def body(y1_ref, p0_ref, p1_ref, c0_ref, c1_ref, w2_ref, o_ref):
        c8 = c0_ref[...] + c1_ref[...]
        row = lax.broadcasted_iota(jnp.int32, (128, 128), 0)
        col = lax.broadcasted_iota(jnp.int32, (128, 128), 1)
        eye = (row == col).astype(jnp.float32)
        for g in range(rb // 128):
            sl = pl.ds(g * 128, 128)
            m = jnp.broadcast_to(c8[g:g + 1, :], (128, 128))
            cnt = jnp.sum(m * eye, axis=-1, keepdims=True)
            agg = (p0_ref[sl, :] + p1_ref[sl, :]) / jnp.maximum(cnt, 1.0)
            o_ref[sl, :] = y1_ref[sl, :] + jnp.dot(
                agg, w2_ref[...], preferred_element_type=jnp.float32)

    grid = (-(-n // rb),)
    return pl.pallas_call(
        body,
        grid=grid,
        in_specs=[
            pl.BlockSpec((rb, d_out), lambda i: (i, 0)),
            pl.BlockSpec((rb, d_in), lambda i: (i, 0)),
            pl.BlockSpec((rb, d_in), lambda i: (i, 0)),
            pl.BlockSpec((rb // 128, 128), lambda i: (i, 0)),
            pl.BlockSpec((rb // 128, 128), lambda i: (i, 0)),
            pl.BlockSpec((d_in, d_out), lambda i: (0, 0)),
        ],
        out_specs=pl.BlockSpec((rb, d_out), lambda i: (i, 0)),
        out_shape=jax.ShapeDtypeStruct((n, d_out), jnp.float32),
    )


@jax.jit
def kernel(node_features, edge_indices, W, b):
    bsz, n, d = node_features.shape
    e = edge_indices.shape[-1]
    d_out = W.shape[0]

    nw = 32
    k = 128
    for cand in range(128, 63, -1):
        if e % cand == 0 and (e // cand) % 8 == 0 and e // cand >= nw:
            k = cand
            break
    n_pad = -(-(n + 64) // 2048) * 2048

    rb = 1
    for cand in range(8, min(n, 1024) + 1, 8):
        if n % cand == 0:
            rb = cand
    if rb == 1:
        rb = n

    n_rows = -(-e // k)
    r8 = -(-n_rows // 8) * 8
    cap = -(-((-(-n_rows // nw) + 1) + 8) // 8) * 8
    cap = min(cap, r8)
    sc_fn = _sc_segment_sum(n_pad, d, nw, n_rows, r8, cap, k)
    tc1_fn = _tc_dense(n, d, d_out, rb)
    tc2_fn = _tc_combine(n, d, d_out)

    w1t = jnp.transpose(W[:, :d])
    w2t = jnp.transpose(W[:, d:])
    b2 = b.reshape(1, d_out)

    outs = []
    for bi in range(bsz):
        ei = edge_indices[bi]
        pad = r8 * k - e
        if pad:
            pad_dst = n + jnp.arange(pad, dtype=jnp.int32) % (n_pad - n)
            ei = jnp.concatenate(
                [ei, jnp.stack([jnp.zeros((pad,), jnp.int32), pad_dst])],
                axis=1)
        ei4 = ei.reshape(1, 2, r8, k)

        x = node_features[bi]
        psum0, psum1, pcnt0, pcnt1 = sc_fn(ei4, x)
        y1 = tc1_fn(x, w1t, b2)
        out = tc2_fn(y1, psum0, psum1,
                     pcnt0.reshape(n_pad // 128, 128),
                     pcnt1.reshape(n_pad // 128, 128), w2t)
        outs.append(out[None])
    return jnp.concatenate(outs, axis=0) if bsz > 1 else outs[0]
